# Initial kernel scaffold; baseline (speedup 1.0000x reference)
import jax
import jax.numpy as jnp
from jax.experimental import pallas as pl


def kernel(x, edge_index, adj_vals, W_weight, W_bias):
    raise NotImplementedError("write your pallas kernel here")


# speedup vs baseline: 3.7845x; 3.7845x over previous
import functools

import jax
import jax.numpy as jnp
from jax import lax
from jax.experimental import pallas as pl
from jax.experimental.pallas import tpu as pltpu
from jax.experimental.pallas import tpu_sc as plsc

N = 10000
E = 320000
NC = 2
NS = 16
NW = NC * NS
EPW = E // NW
C = 80
NCH = EPW // C
RPT = 624
ZR = 208


def _make_spmm(d, tc_tiling=True):
  mesh = plsc.VectorSubcoreMesh(core_axis_name="c", subcore_axis_name="s")

  @functools.partial(
      pl.kernel,
      out_type=jax.ShapeDtypeStruct((2 * N, d), jnp.float32),
      mesh=mesh,
      compiler_params=pltpu.CompilerParams(
          needs_layout_passes=False, use_tc_tiling_on_sc=tc_tiling),
      scratch_types=[
          pltpu.VMEM((C,), jnp.int32),
          pltpu.VMEM((C,), jnp.int32),
          pltpu.VMEM((C,), jnp.float32),
          pltpu.VMEM((C, d), jnp.float32),
          pltpu.VMEM((ZR, d), jnp.float32),
          pltpu.VMEM_SHARED((N, d), jnp.float32),
          pltpu.SemaphoreType.DMA,
      ],
  )
  def spmm(mat_hbm, row_hbm, col_hbm, vals_hbm, out_hbm,
           col_v, row_v, vals_v, g_v, z_v, acc, sem):
    cid = lax.axis_index("c")
    sid = lax.axis_index("s")
    wid = cid * NS + sid

    zero16 = jnp.zeros((16,), jnp.float32)

    def zrow(i, carry):
      for j in range(d // 16):
        z_v[i, pl.ds(j * 16, 16)] = zero16
      return carry

    lax.fori_loop(0, ZR, zrow, 0)
    for k in range(RPT // ZR):
      pltpu.sync_copy(z_v, acc.at[pl.ds(sid * RPT + k * ZR, ZR)])

    @pl.when(sid == NS - 1)
    def _():
      pltpu.sync_copy(z_v.at[pl.ds(0, N - NS * RPT)],
                      acc.at[pl.ds(NS * RPT, N - NS * RPT)])

    plsc.subcore_barrier()

    def chunk_body(c, carry):
      base = wid * EPW + c * C
      pltpu.sync_copy(row_hbm.at[pl.ds(base, C)], row_v)
      pltpu.sync_copy(col_hbm.at[pl.ds(base, C)], col_v)
      pltpu.sync_copy(vals_hbm.at[pl.ds(base, C)], vals_v)
      pltpu.async_copy(mat_hbm.at[col_v], g_v, sem).wait()

      def scale(i, carry2):
        bv = plsc.load_gather(vals_v, [jnp.full((16,), i, jnp.int32)])
        for j in range(d // 16):
          g_v[i, pl.ds(j * 16, 16)] = g_v[i, pl.ds(j * 16, 16)] * bv
        return carry2

      lax.fori_loop(0, C, scale, 0)
      pltpu.sync_copy(g_v, acc.at[row_v], add=True)
      return carry

    lax.fori_loop(0, NCH, chunk_body, 0)

    plsc.subcore_barrier()
    pltpu.sync_copy(acc.at[pl.ds(sid * RPT, RPT)],
                    out_hbm.at[pl.ds(cid * N + sid * RPT, RPT)])

    @pl.when(sid == NS - 1)
    def _():
      pltpu.sync_copy(acc.at[pl.ds(NS * RPT, N - NS * RPT)],
                      out_hbm.at[pl.ds(cid * N + NS * RPT, N - NS * RPT)])

  return spmm


_spmm128 = _make_spmm(128)
_spmm64 = _make_spmm(64, tc_tiling=False)

_BR = 1000
_NB = N // _BR


def _combine(p, d, relu):
  def body(a_ref, b_ref, o_ref):
    s = a_ref[...] + b_ref[...]
    o_ref[...] = jnp.maximum(s, 0.0) if relu else s

  return pl.pallas_call(
      body,
      grid=(_NB,),
      in_specs=[
          pl.BlockSpec((_BR, d), lambda i: (i, 0)),
          pl.BlockSpec((_BR, d), lambda i: (_NB + i, 0)),
      ],
      out_specs=pl.BlockSpec((_BR, d), lambda i: (i, 0)),
      out_shape=jax.ShapeDtypeStruct((N, d), jnp.float32),
  )(p, p)


def _combine_relu_linear(q, w, bias):
  def body(a_ref, b_ref, w_ref, bias_ref, o_ref):
    h = jnp.maximum(a_ref[...] + b_ref[...], 0.0)
    o_ref[...] = (
        jnp.dot(h, w_ref[...].T, preferred_element_type=jnp.float32)
        + bias_ref[...]
    )

  return pl.pallas_call(
      body,
      grid=(_NB,),
      in_specs=[
          pl.BlockSpec((_BR, 128), lambda i: (i, 0)),
          pl.BlockSpec((_BR, 128), lambda i: (_NB + i, 0)),
          pl.BlockSpec((64, 128), lambda i: (0, 0)),
          pl.BlockSpec((1, 64), lambda i: (0, 0)),
      ],
      out_specs=pl.BlockSpec((_BR, 64), lambda i: (i, 0)),
      out_shape=jax.ShapeDtypeStruct((N, 64), jnp.float32),
  )(q, q, w, bias)


def kernel(x, edge_index, adj_vals, W_weight, W_bias):
  row = edge_index[0]
  col = edge_index[1]
  bias2d = W_bias.reshape(1, 64)

  p = _spmm128(x, row, col, adj_vals)
  h1 = _combine(p, 128, relu=True)
  q = _spmm128(h1, row, col, adj_vals)
  support = _combine_relu_linear(q, W_weight, bias2d)
  r = _spmm64(support, row, col, adj_vals)
  out = _combine(r, 64, relu=False)
  return out


# speedup vs baseline: 11.3895x; 3.0095x over previous
import functools

import jax
import jax.numpy as jnp
from jax import lax
from jax.experimental import pallas as pl
from jax.experimental.pallas import tpu as pltpu
from jax.experimental.pallas import tpu_sc as plsc

N = 10000
E = 320000
NC = 2
NS = 16
NW = NC * NS
C = 128
NCHUNK = E // C
KFULL = NCHUNK // NW
KEXTRA = NCHUNK % NW
RPT = 624


def _make_spmm(d, tc_tiling=True):
  mesh = plsc.VectorSubcoreMesh(core_axis_name="c", subcore_axis_name="s")
  nb = d // 16

  @functools.partial(
      pl.kernel,
      out_type=jax.ShapeDtypeStruct((2 * N, d), jnp.float32),
      mesh=mesh,
      compiler_params=pltpu.CompilerParams(
          needs_layout_passes=False, use_tc_tiling_on_sc=tc_tiling),
      scratch_types=[
          [pltpu.VMEM((C,), jnp.int32) for _ in range(3)],
          [pltpu.VMEM((C,), jnp.int32) for _ in range(3)],
          [pltpu.VMEM((C,), jnp.float32) for _ in range(3)],
          [pltpu.VMEM((C,), jnp.int32) for _ in range(3)],
          [pltpu.VMEM((C, d), jnp.float32) for _ in range(3)],
          pltpu.VMEM_SHARED((N, d), jnp.float32),
          [pltpu.SemaphoreType.DMA for _ in range(3)],
          [pltpu.SemaphoreType.DMA for _ in range(3)],
          [pltpu.SemaphoreType.DMA for _ in range(3)],
      ],
  )
  def spmm(mat_hbm, row_hbm, col_hbm, vals_hbm, out_hbm,
           colv, rowv, valsv, rsc, g, acc, si, sg, ss):
    cid = lax.axis_index("c")
    sid = lax.axis_index("s")
    wid = cid * NS + sid

    def base_of(cloc):
      return jnp.minimum((wid + NW * cloc) * C, E - C)

    def fire_idx(cloc, b):
      base = base_of(cloc)
      pltpu.async_copy(row_hbm.at[pl.ds(base, C)], rowv[b], si[b])
      pltpu.async_copy(col_hbm.at[pl.ds(base, C)], colv[b], si[b])
      pltpu.async_copy(vals_hbm.at[pl.ds(base, C)], valsv[b], si[b])

    def wait_idx(b):
      pltpu.make_async_copy(row_hbm.at[pl.ds(0, C)], rowv[b], si[b]).wait()
      pltpu.make_async_copy(col_hbm.at[pl.ds(0, C)], colv[b], si[b]).wait()
      pltpu.make_async_copy(vals_hbm.at[pl.ds(0, C)], valsv[b], si[b]).wait()

    def fire_gather(b):
      pltpu.async_copy(mat_hbm.at[colv[b]], g[b], sg[b])

    def wait_gather(b):
      pltpu.make_async_copy(mat_hbm.at[colv[b]], g[b], sg[b]).wait()

    def fire_scatter(b):
      pltpu.async_copy(g[b], acc.at[rsc[b]], ss[b], add=True)

    def wait_scatter(b):
      pltpu.make_async_copy(g[b], acc.at[rsc[b]], ss[b]).wait()

    def copy_row_to_rsc(b):
      for t in range(C // 16):
        rsc[b][pl.ds(t * 16, 16)] = rowv[b][pl.ds(t * 16, 16)]

    def scale(b):
      def body(i4, carry):
        for u in range(4):
          i = i4 * 4 + u
          bv = plsc.load_gather(valsv[b], [jnp.full((16,), i, jnp.int32)])
          for t in range(nb):
            sl = pl.ds(t * 16, 16)
            g[b][i, sl] = g[b][i, sl] * bv
        return carry

      lax.fori_loop(0, C // 4, body, 0)

    zero16f = jnp.zeros((16,), jnp.float32)
    zero16i = jnp.zeros((16,), jnp.int32)

    def grow(i, carry):
      for b in range(3):
        for t in range(nb):
          g[b][i, pl.ds(t * 16, 16)] = zero16f
      return carry

    lax.fori_loop(0, C, grow, 0)
    for b in range(3):
      for t in range(C // 16):
        rsc[b][pl.ds(t * 16, 16)] = zero16i

    for k in range(RPT // C):
      pltpu.sync_copy(g[0], acc.at[pl.ds(sid * RPT + k * C, C)])
    rem = RPT - (RPT // C) * C
    pltpu.sync_copy(g[0].at[pl.ds(0, rem)],
                    acc.at[pl.ds(sid * RPT + (RPT // C) * C, rem)])

    @pl.when(sid == NS - 1)
    def _():
      pltpu.sync_copy(g[0].at[pl.ds(0, N - NS * RPT)],
                      acc.at[pl.ds(NS * RPT, N - NS * RPT)])

    plsc.subcore_barrier()

    fire_idx(0, 0)
    fire_idx(1, 1)
    fire_idx(2, 2)
    fire_scatter(1)
    fire_scatter(2)
    wait_idx(0)
    fire_gather(0)

    def phase(k, j):
      b = j
      bn = (j + 1) % 3
      cloc = 3 * k + j
      wait_gather(b)
      copy_row_to_rsc(b)
      wait_scatter(bn)
      wait_idx(bn)
      fire_gather(bn)
      scale(b)
      fire_scatter(b)
      fire_idx(cloc + 3, b)

    def iteration(k, carry):
      phase(k, 0)
      phase(k, 1)
      phase(k, 2)
      return carry

    lax.fori_loop(0, KFULL // 3, iteration, 0)

    wait_gather(0)

    @pl.when(wid < KEXTRA)
    def _():
      copy_row_to_rsc(0)
      scale(0)
      pltpu.sync_copy(g[0], acc.at[rsc[0]], add=True)

    wait_idx(1)
    wait_idx(2)
    wait_scatter(1)
    wait_scatter(2)

    plsc.subcore_barrier()
    pltpu.sync_copy(acc.at[pl.ds(sid * RPT, RPT)],
                    out_hbm.at[pl.ds(cid * N + sid * RPT, RPT)])

    @pl.when(sid == NS - 1)
    def _():
      pltpu.sync_copy(acc.at[pl.ds(NS * RPT, N - NS * RPT)],
                      out_hbm.at[pl.ds(cid * N + NS * RPT, N - NS * RPT)])

  return spmm


_spmm128 = _make_spmm(128)
_spmm64 = _make_spmm(64, tc_tiling=False)

_BR = 1000
_NB = N // _BR


def _combine(p, d, relu):
  def body(a_ref, b_ref, o_ref):
    s = a_ref[...] + b_ref[...]
    o_ref[...] = jnp.maximum(s, 0.0) if relu else s

  return pl.pallas_call(
      body,
      grid=(_NB,),
      in_specs=[
          pl.BlockSpec((_BR, d), lambda i: (i, 0)),
          pl.BlockSpec((_BR, d), lambda i: (_NB + i, 0)),
      ],
      out_specs=pl.BlockSpec((_BR, d), lambda i: (i, 0)),
      out_shape=jax.ShapeDtypeStruct((N, d), jnp.float32),
  )(p, p)


def _combine_relu_linear(q, w, bias):
  def body(a_ref, b_ref, w_ref, bias_ref, o_ref):
    h = jnp.maximum(a_ref[...] + b_ref[...], 0.0)
    o_ref[...] = (
        jnp.dot(h, w_ref[...].T, preferred_element_type=jnp.float32)
        + bias_ref[...]
    )

  return pl.pallas_call(
      body,
      grid=(_NB,),
      in_specs=[
          pl.BlockSpec((_BR, 128), lambda i: (i, 0)),
          pl.BlockSpec((_BR, 128), lambda i: (_NB + i, 0)),
          pl.BlockSpec((64, 128), lambda i: (0, 0)),
          pl.BlockSpec((1, 64), lambda i: (0, 0)),
      ],
      out_specs=pl.BlockSpec((_BR, 64), lambda i: (i, 0)),
      out_shape=jax.ShapeDtypeStruct((N, 64), jnp.float32),
  )(q, q, w, bias)


def kernel(x, edge_index, adj_vals, W_weight, W_bias):
  row = edge_index[0]
  col = edge_index[1]
  bias2d = W_bias.reshape(1, 64)

  p = _spmm128(x, row, col, adj_vals)
  h1 = _combine(p, 128, relu=True)
  q = _spmm128(h1, row, col, adj_vals)
  support = _combine_relu_linear(q, W_weight, bias2d)
  r = _spmm64(support, row, col, adj_vals)
  out = _combine(r, 64, relu=False)
  return out


# speedup vs baseline: 13.3570x; 1.1727x over previous
import functools

import jax
import jax.numpy as jnp
import numpy as np
from jax import lax
from jax.experimental import pallas as pl
from jax.experimental.pallas import tpu as pltpu
from jax.experimental.pallas import tpu_sc as plsc

N = 10000
E = 320000
NC = 2
NS = 16
NW = NC * NS
C = 128
NCHUNK = E // C
KFULL = NCHUNK // NW
KEXTRA = NCHUNK % NW
RPT = 624


def _perm(d):
  p = np.empty(d, np.int32)
  for g in range(d // 32):
    for i in range(16):
      p[32 * g + 2 * i] = 32 * g + i
      p[32 * g + 2 * i + 1] = 32 * g + 16 + i
  return p


_P128 = _perm(128)
_P64 = _perm(64)
_PMAT128 = np.eye(128, dtype=np.float32)[_P128].T


def _make_spmm(d):
  mesh = plsc.VectorSubcoreMesh(core_axis_name="c", subcore_axis_name="s")
  ng = d // 32

  @functools.partial(
      pl.kernel,
      out_type=jax.ShapeDtypeStruct((2 * N, d), jnp.float32),
      mesh=mesh,
      compiler_params=pltpu.CompilerParams(
          needs_layout_passes=False, use_tc_tiling_on_sc=False),
      scratch_types=[
          [pltpu.VMEM((C,), jnp.int32) for _ in range(3)],
          [pltpu.VMEM((C,), jnp.int32) for _ in range(3)],
          [pltpu.VMEM((C,), jnp.float32) for _ in range(3)],
          [pltpu.VMEM((C,), jnp.int32) for _ in range(2)],
          [pltpu.VMEM((C, d), jnp.bfloat16) for _ in range(2)],
          [pltpu.VMEM((C, d), jnp.float32) for _ in range(2)],
          pltpu.VMEM_SHARED((N, d), jnp.float32),
          [pltpu.SemaphoreType.DMA for _ in range(3)],
          [pltpu.SemaphoreType.DMA for _ in range(2)],
          [pltpu.SemaphoreType.DMA for _ in range(2)],
      ],
  )
  def spmm(mat_hbm, ei_hbm, vals_hbm, out_hbm,
           colv, rowv, valsv, rsc, gb, gf, acc, si, sg, ss):
    cid = lax.axis_index("c")
    sid = lax.axis_index("s")
    wid = cid * NS + sid

    def base_of(cloc):
      return jnp.minimum((wid + NW * cloc) * C, E - C)

    def fire_idx(cloc, j):
      base = base_of(cloc)
      pltpu.async_copy(ei_hbm.at[0, pl.ds(base, C)], rowv[j], si[j])
      pltpu.async_copy(ei_hbm.at[1, pl.ds(base, C)], colv[j], si[j])
      pltpu.async_copy(vals_hbm.at[pl.ds(base, C)], valsv[j], si[j])

    def wait_idx(j):
      pltpu.make_async_copy(ei_hbm.at[0, pl.ds(0, C)], rowv[j], si[j]).wait()
      pltpu.make_async_copy(ei_hbm.at[1, pl.ds(0, C)], colv[j], si[j]).wait()
      pltpu.make_async_copy(vals_hbm.at[pl.ds(0, C)], valsv[j], si[j]).wait()

    def fire_gather(j, b):
      pltpu.async_copy(mat_hbm.at[colv[j]], gb[b], sg[b])

    def wait_gather(j, b):
      pltpu.make_async_copy(mat_hbm.at[colv[j]], gb[b], sg[b]).wait()

    def fire_scatter(b):
      pltpu.async_copy(gf[b], acc.at[rsc[b]], ss[b], add=True)

    def wait_scatter(b):
      pltpu.make_async_copy(gf[b], acc.at[rsc[b]], ss[b]).wait()

    def copy_row_to_rsc(j, b):
      for t in range(C // 16):
        rsc[b][pl.ds(t * 16, 16)] = rowv[j][pl.ds(t * 16, 16)]

    mask_hi = jnp.full((16,), -65536, jnp.int32)

    def scale(j, b):
      @plsc.parallel_loop(0, C, step=2, unroll=2)
      def body(i2):
        for u in range(2):
          i = i2 + u
          bv = plsc.load_gather(valsv[j], [jnp.full((16,), i, jnp.int32)])
          for t in range(ng):
            xi = plsc.bitcast(gb[b][i, pl.ds(t * 32, 32)], jnp.int32)
            lo = plsc.bitcast(xi << 16, jnp.float32)
            hi = plsc.bitcast(xi & mask_hi, jnp.float32)
            gf[b][i, pl.ds(t * 32, 16)] = lo * bv
            gf[b][i, pl.ds(t * 32 + 16, 16)] = hi * bv

    zero16f = jnp.zeros((16,), jnp.float32)
    zero16i = jnp.zeros((16,), jnp.int32)

    def grow(i, carry):
      for b in range(2):
        for t in range(d // 16):
          gf[b][i, pl.ds(t * 16, 16)] = zero16f
      return carry

    lax.fori_loop(0, C, grow, 0)
    for b in range(2):
      for t in range(C // 16):
        rsc[b][pl.ds(t * 16, 16)] = zero16i

    for k in range(RPT // C):
      pltpu.sync_copy(gf[0], acc.at[pl.ds(sid * RPT + k * C, C)])
    rem = RPT - (RPT // C) * C
    pltpu.sync_copy(gf[0].at[pl.ds(0, rem)],
                    acc.at[pl.ds(sid * RPT + (RPT // C) * C, rem)])

    @pl.when(sid == NS - 1)
    def _():
      pltpu.sync_copy(gf[0].at[pl.ds(0, N - NS * RPT)],
                      acc.at[pl.ds(NS * RPT, N - NS * RPT)])

    plsc.subcore_barrier()

    fire_idx(0, 0)
    fire_idx(1, 1)
    fire_idx(2, 2)
    fire_scatter(0)
    fire_scatter(1)
    wait_idx(0)
    fire_gather(0, 0)

    def phase(k, u):
      cloc = 6 * k + u
      j = u % 3
      jn = (u + 1) % 3
      b = u % 2
      bn = (u + 1) % 2
      wait_gather(j, b)
      wait_idx(jn)
      fire_gather(jn, bn)
      wait_scatter(b)
      copy_row_to_rsc(j, b)
      scale(j, b)
      fire_scatter(b)
      fire_idx(cloc + 3, j)
      return cloc

    def iteration(k, carry):
      for u in range(6):
        phase(k, u)
      return carry

    lax.fori_loop(0, KFULL // 6, iteration, 0)

    wait_gather(0, 0)
    wait_scatter(0)
    wait_scatter(1)

    @pl.when(wid < KEXTRA)
    def _():
      copy_row_to_rsc(0, 0)
      scale(0, 0)
      pltpu.sync_copy(gf[0], acc.at[rsc[0]], add=True)

    wait_idx(1)
    wait_idx(2)

    plsc.subcore_barrier()
    pltpu.sync_copy(acc.at[pl.ds(sid * RPT, RPT)],
                    out_hbm.at[pl.ds(cid * N + sid * RPT, RPT)])

    @pl.when(sid == NS - 1)
    def _():
      pltpu.sync_copy(acc.at[pl.ds(NS * RPT, N - NS * RPT)],
                      out_hbm.at[pl.ds(cid * N + NS * RPT, N - NS * RPT)])

  return spmm


_spmm128 = _make_spmm(128)
_spmm64 = _make_spmm(64)

_BR = 1000
_NB = N // _BR


def _combine_relu_perm(p):
  def body(a_ref, b_ref, m_ref, o_ref):
    h = jnp.maximum(a_ref[...] + b_ref[...], 0.0)
    o_ref[...] = jnp.dot(
        h, m_ref[...], preferred_element_type=jnp.float32
    ).astype(jnp.bfloat16)

  return pl.pallas_call(
      body,
      grid=(_NB,),
      in_specs=[
          pl.BlockSpec((_BR, 128), lambda i: (i, 0)),
          pl.BlockSpec((_BR, 128), lambda i: (_NB + i, 0)),
          pl.BlockSpec((128, 128), lambda i: (0, 0)),
      ],
      out_specs=pl.BlockSpec((_BR, 128), lambda i: (i, 0)),
      out_shape=jax.ShapeDtypeStruct((N, 128), jnp.bfloat16),
  )(p, p, jnp.asarray(_PMAT128))


def _combine_relu_linear(q, wp, biasp):
  def body(a_ref, b_ref, w_ref, bias_ref, o_ref):
    h = jnp.maximum(a_ref[...] + b_ref[...], 0.0)
    o_ref[...] = (
        jnp.dot(h, w_ref[...].T, preferred_element_type=jnp.float32)
        + bias_ref[...]
    ).astype(jnp.bfloat16)

  return pl.pallas_call(
      body,
      grid=(_NB,),
      in_specs=[
          pl.BlockSpec((_BR, 128), lambda i: (i, 0)),
          pl.BlockSpec((_BR, 128), lambda i: (_NB + i, 0)),
          pl.BlockSpec((64, 128), lambda i: (0, 0)),
          pl.BlockSpec((1, 64), lambda i: (0, 0)),
      ],
      out_specs=pl.BlockSpec((_BR, 64), lambda i: (i, 0)),
      out_shape=jax.ShapeDtypeStruct((N, 64), jnp.bfloat16),
  )(q, q, wp, biasp)


def _combine_final(r):
  def body(a_ref, b_ref, o_ref):
    o_ref[...] = a_ref[...] + b_ref[...]

  return pl.pallas_call(
      body,
      grid=(_NB,),
      in_specs=[
          pl.BlockSpec((_BR, 64), lambda i: (i, 0)),
          pl.BlockSpec((_BR, 64), lambda i: (_NB + i, 0)),
      ],
      out_specs=pl.BlockSpec((_BR, 64), lambda i: (i, 0)),
      out_shape=jax.ShapeDtypeStruct((N, 64), jnp.float32),
  )(r, r)


def kernel(x, edge_index, adj_vals, W_weight, W_bias):
  xp = x[:, _P128].astype(jnp.bfloat16)
  wp = W_weight[_P64]
  biasp = W_bias[_P64].reshape(1, 64)

  p = _spmm128(xp, edge_index, adj_vals)
  h1 = _combine_relu_perm(p)
  q = _spmm128(h1, edge_index, adj_vals)
  support = _combine_relu_linear(q, wp, biasp)
  r = _spmm64(support, edge_index, adj_vals)
  return _combine_final(r)
